# baseline (speedup 1.0000x reference)
import jax
import jax.numpy as jnp
from jax import lax
from jax.experimental import pallas as pl
from jax.experimental.pallas import tpu as pltpu
from jax.experimental.pallas import tpu_sc as plsc

_B = 64
_V = 100000

_G = 512
_NSUB = 32
_C_SC = _G * _NSUB
_C_TC = _V - _C_SC

_BLK = 8192
_CHUNK = 2048

_K0 = 0
_K1 = 42
_K2 = _K0 ^ _K1 ^ 0x1BD11BDA

_ROTS = ((13, 15, 26, 6), (17, 29, 16, 24))
_SCHED = ((_K1, _K2), (_K2, _K0), (_K0, _K1), (_K1, _K2), (_K2, _K0))

_LN2 = 0.6931471805599453


def _threefry_bits(x1):
    x1 = x1 + jnp.uint32(_K1)
    x0 = x1
    x1 = ((x1 << jnp.uint32(13)) | (x1 >> jnp.uint32(19))) ^ x0
    first = True
    for i in range(5):
        for r in _ROTS[i % 2]:
            if first:
                first = False
                continue
            x0 = x0 + x1
            x1 = (x1 << jnp.uint32(r)) | (x1 >> jnp.uint32(32 - r))
            x1 = x1 ^ x0
        ka, kb = _SCHED[i]
        x0 = x0 + jnp.uint32(ka)
        x1 = x1 + jnp.uint32(kb) + jnp.uint32(i + 1)
    return x0 ^ x1


def _bits_to_uniform(bits):
    fbits = (bits >> jnp.uint32(9)) | jnp.uint32(0x3F800000)
    u = lax.bitcast_convert_type(fbits, jnp.float32) - 1.0
    return jnp.maximum(u, jnp.float32(1e-20))


def _tc_kernel(x_ref, oi_ref, om_ref, max_ref, idx_ref):
    b = pl.program_id(0)
    nb = pl.num_programs(0)

    @pl.when(b == 0)
    def _init():
        max_ref[...] = jnp.full_like(max_ref[...], -jnp.inf)
        idx_ref[...] = jnp.zeros_like(idx_ref[...])

    acc_m = max_ref[...]
    acc_i = idx_ref[...]
    for c in range(_BLK // _CHUNK):
        shp = (_B, _CHUNK)
        col = (lax.broadcasted_iota(jnp.int32, shp, 1)
               + (b * _BLK + c * _CHUNK + _C_SC))
        row = lax.broadcasted_iota(jnp.uint32, shp, 0)
        flat = row * jnp.uint32(_V) + col.astype(jnp.uint32)

        bits = _threefry_bits(flat)
        u = _bits_to_uniform(bits)
        gumbel = -jnp.log(-jnp.log(u))

        pert = x_ref[:, c * _CHUNK:(c + 1) * _CHUNK] + gumbel
        pert = jnp.where(col < _V, pert, -jnp.inf)

        for s in range(_CHUNK // 128):
            tile = pert[:, s * 128:(s + 1) * 128]
            itile = col[:, s * 128:(s + 1) * 128]
            upd = tile > acc_m
            acc_i = jnp.where(upd, itile, acc_i)
            acc_m = jnp.maximum(acc_m, tile)
    max_ref[...] = acc_m
    idx_ref[...] = acc_i

    @pl.when(b == nb - 1)
    def _done():
        m = jnp.max(acc_m, axis=1, keepdims=True)
        cand = jnp.where(acc_m == m, acc_i, jnp.int32(0x7FFFFFFF))
        oi_ref[...] = jnp.min(cand, axis=1, keepdims=True)
        om_ref[...] = m


def _neg_ln(u):
    bits = lax.bitcast_convert_type(u, jnp.uint32)
    e = (bits >> jnp.uint32(23)).astype(jnp.int32) - 127
    m = lax.bitcast_convert_type(
        (bits & jnp.uint32(0x7FFFFF)) | jnp.uint32(0x3F800000), jnp.float32)
    big = m > jnp.float32(1.4142135)
    m = jnp.where(big, m * jnp.float32(0.5), m)
    e = e + jnp.where(big, jnp.int32(1), jnp.int32(0))
    ef = e.astype(jnp.float32)
    z = (m - 1.0) / (m + 1.0)
    z2 = z * z
    p = jnp.float32(2 / 9)
    for cc in (2 / 7, 2 / 5, 2 / 3, 2.0):
        p = p * z2 + jnp.float32(cc)
    lnm = z * p
    return -(ef * jnp.float32(_LN2) + lnm)


def _sc_body(x_ref, ol_ref, oi_ref,
             stripe, bl_vmem, bi_vmem, sem_in, sem_l, sem_i):
    c = lax.axis_index("c")
    s = lax.axis_index("s")
    t = c * 16 + s
    col0 = t * _G

    cp = pltpu.make_async_copy(x_ref.at[:, pl.ds(col0, _G)], stripe, sem_in)
    cp.start()
    cp.wait()

    iota_i = lax.iota(jnp.int32, 16)
    iota_u = lax.iota(jnp.uint32, 16)

    def row_body(r, _):
        base_u = jnp.uint32(r * _V + col0)

        def col_body(v, carry):
            bn, bd, bi, blg = carry
            l = stripe[r, pl.ds(v * 16, 16)]
            colv = (col0 + v * 16) + iota_i
            flat = (base_u + jnp.uint32(v * 16)) + iota_u
            bits = _threefry_bits(flat)
            u = _bits_to_uniform(bits)
            y = _neg_ln(u)
            num = jnp.exp(l)
            upd = num * bd > bn * y
            bn = jnp.where(upd, num, bn)
            bd = jnp.where(upd, y, bd)
            bi = jnp.where(upd, colv, bi)
            blg = jnp.where(upd, l, blg)
            return bn, bd, bi, blg

        init = (jnp.zeros(16, jnp.float32), jnp.ones(16, jnp.float32),
                jnp.zeros(16, jnp.int32), jnp.zeros(16, jnp.float32))
        bn, bd, bi, blg = lax.fori_loop(0, _G // 16, col_body, init)
        bl_vmem[r] = blg
        bi_vmem[r] = bi
        return 0

    lax.fori_loop(0, _B, row_body, 0)

    cpl = pltpu.make_async_copy(bl_vmem, ol_ref.at[t], sem_l)
    cpl.start()
    cpi = pltpu.make_async_copy(bi_vmem, oi_ref.at[t], sem_i)
    cpi.start()
    cpl.wait()
    cpi.wait()


def _sc_candidates(logits):
    fn = pl.kernel(
        _sc_body,
        out_type=(
            jax.ShapeDtypeStruct((_NSUB, _B, 16), jnp.float32),
            jax.ShapeDtypeStruct((_NSUB, _B, 16), jnp.int32),
        ),
        mesh=plsc.VectorSubcoreMesh(core_axis_name="c", subcore_axis_name="s",
                                    num_cores=2, num_subcores=16),
        scratch_types=(
            pltpu.VMEM((_B, _G), jnp.float32),
            pltpu.VMEM((_B, 16), jnp.float32),
            pltpu.VMEM((_B, 16), jnp.int32),
            pltpu.SemaphoreType.DMA,
            pltpu.SemaphoreType.DMA,
            pltpu.SemaphoreType.DMA,
        ),
    )
    return fn(logits)


def _merge_kernel(cl_ref, ci_ref, tm_ref, ti_ref, o_ref):
    ci = ci_ref[...]
    row = lax.broadcasted_iota(jnp.uint32, ci.shape, 0)
    flat = row * jnp.uint32(_V) + ci.astype(jnp.uint32)
    bits = _threefry_bits(flat)
    u = _bits_to_uniform(bits)
    gumbel = -jnp.log(-jnp.log(u))
    sc = cl_ref[...] + gumbel

    m = jnp.max(sc, axis=1, keepdims=True)
    cand = jnp.where(sc == m, ci, jnp.int32(0x7FFFFFFF))
    ii = jnp.min(cand, axis=1, keepdims=True)
    better = m >= tm_ref[...]
    o_ref[...] = jnp.where(better, ii, ti_ref[...])


def kernel(logits):
    tc_idx, tc_max = pl.pallas_call(
        _tc_kernel,
        grid=(pl.cdiv(_C_TC, _BLK),),
        in_specs=[pl.BlockSpec((_B, _BLK), lambda b: (0, b + _C_SC // _BLK))],
        out_specs=(pl.BlockSpec((_B, 1), lambda b: (0, 0)),
                   pl.BlockSpec((_B, 1), lambda b: (0, 0))),
        out_shape=(jax.ShapeDtypeStruct((_B, 1), jnp.int32),
                   jax.ShapeDtypeStruct((_B, 1), jnp.float32)),
        scratch_shapes=[
            pltpu.VMEM((_B, 128), jnp.float32),
            pltpu.VMEM((_B, 128), jnp.int32),
        ],
    )(logits)

    sc_l, sc_i = _sc_candidates(logits)
    sc_l = sc_l.transpose(1, 0, 2).reshape(_B, _NSUB * 16)
    sc_i = sc_i.transpose(1, 0, 2).reshape(_B, _NSUB * 16)

    out = pl.pallas_call(
        _merge_kernel,
        out_shape=jax.ShapeDtypeStruct((_B, 1), jnp.int32),
    )(sc_l, sc_i, tc_max, tc_idx)
    return out[:, 0].astype(jnp.int64)


# speedup vs baseline: 1.0591x; 1.0591x over previous
import jax
import jax.numpy as jnp
from jax import lax
from jax.experimental import pallas as pl
from jax.experimental.pallas import tpu as pltpu
from jax.experimental.pallas import tpu_sc as plsc

_B = 64
_V = 100000

_G = 768
_NSUB = 32
_C_SC = _G * _NSUB
_C_TC = _V - _C_SC

_BLK = 8192
_CHUNK = 2048

_K0 = 0
_K1 = 42
_K2 = _K0 ^ _K1 ^ 0x1BD11BDA

_ROTS = ((13, 15, 26, 6), (17, 29, 16, 24))
_SCHED = ((_K1, _K2), (_K2, _K0), (_K0, _K1), (_K1, _K2), (_K2, _K0))

_LN2 = 0.6931471805599453


def _threefry_bits(x1):
    x1 = x1 + jnp.uint32(_K1)
    x0 = x1
    x1 = ((x1 << jnp.uint32(13)) | (x1 >> jnp.uint32(19))) ^ x0
    first = True
    for i in range(5):
        for r in _ROTS[i % 2]:
            if first:
                first = False
                continue
            x0 = x0 + x1
            x1 = (x1 << jnp.uint32(r)) | (x1 >> jnp.uint32(32 - r))
            x1 = x1 ^ x0
        ka, kb = _SCHED[i]
        x0 = x0 + jnp.uint32(ka)
        x1 = x1 + jnp.uint32(kb) + jnp.uint32(i + 1)
    return x0 ^ x1


def _bits_to_uniform(bits):
    fbits = (bits >> jnp.uint32(9)) | jnp.uint32(0x3F800000)
    u = lax.bitcast_convert_type(fbits, jnp.float32) - 1.0
    return jnp.maximum(u, jnp.float32(1e-20))


def _tc_kernel(x_ref, oi_ref, om_ref, max_ref, idx_ref):
    b = pl.program_id(0)
    nb = pl.num_programs(0)

    @pl.when(b == 0)
    def _init():
        max_ref[...] = jnp.full_like(max_ref[...], -jnp.inf)
        idx_ref[...] = jnp.zeros_like(idx_ref[...])

    acc_m = max_ref[...]
    acc_i = idx_ref[...]
    shp = (_B, _CHUNK)
    base2d = (lax.broadcasted_iota(jnp.uint32, shp, 0) * jnp.uint32(_V)
              + lax.broadcasted_iota(jnp.uint32, shp, 1))
    iota2d = lax.broadcasted_iota(jnp.int32, shp, 1)
    for c in range(_BLK // _CHUNK):
        off = b * _BLK + c * _CHUNK + _C_SC
        col = iota2d + off
        flat = base2d + jnp.uint32(off)

        bits = _threefry_bits(flat)
        u = _bits_to_uniform(bits)
        gumbel = -jnp.log(-jnp.log(u))

        pert = x_ref[:, c * _CHUNK:(c + 1) * _CHUNK] + gumbel
        pert = jnp.where(col < _V, pert, -jnp.inf)

        for s in range(_CHUNK // 128):
            tile = pert[:, s * 128:(s + 1) * 128]
            itile = col[:, s * 128:(s + 1) * 128]
            upd = tile > acc_m
            acc_i = jnp.where(upd, itile, acc_i)
            acc_m = jnp.maximum(acc_m, tile)
    max_ref[...] = acc_m
    idx_ref[...] = acc_i

    @pl.when(b == nb - 1)
    def _done():
        m = jnp.max(acc_m, axis=1, keepdims=True)
        cand = jnp.where(acc_m == m, acc_i, jnp.int32(0x7FFFFFFF))
        oi_ref[...] = jnp.min(cand, axis=1, keepdims=True)
        om_ref[...] = m


def _neg_ln(u):
    bits = lax.bitcast_convert_type(u, jnp.uint32)
    e = (bits >> jnp.uint32(23)).astype(jnp.int32) - 127
    m = lax.bitcast_convert_type(
        (bits & jnp.uint32(0x7FFFFF)) | jnp.uint32(0x3F800000), jnp.float32)
    big = m > jnp.float32(1.4142135)
    m = jnp.where(big, m * jnp.float32(0.5), m)
    e = e + jnp.where(big, jnp.int32(1), jnp.int32(0))
    ef = e.astype(jnp.float32)
    z = (m - 1.0) / (m + 1.0)
    z2 = z * z
    p = jnp.float32(2 / 9)
    for cc in (2 / 7, 2 / 5, 2 / 3, 2.0):
        p = p * z2 + jnp.float32(cc)
    lnm = z * p
    return -(ef * jnp.float32(_LN2) + lnm)


def _sc_body(x_ref, ol_ref, oi_ref,
             stripe, bl_vmem, bi_vmem, sem_in, sem_l, sem_i):
    c = lax.axis_index("c")
    s = lax.axis_index("s")
    t = c * 16 + s
    col0 = t * _G

    cp = pltpu.make_async_copy(x_ref.at[:, pl.ds(col0, _G)], stripe, sem_in)
    cp.start()
    cp.wait()

    iota_i = lax.iota(jnp.int32, 16)
    iota_u = lax.iota(jnp.uint32, 16)

    def row_body(r, _):
        base_u = jnp.uint32(r * _V + col0)

        def col_body(v, carry):
            bn, bd, bi, blg = carry
            for k in range(2):
                o = v * 32 + k * 16
                l = stripe[r, pl.ds(o, 16)]
                colv = (col0 + o) + iota_i
                flat = (base_u + jnp.uint32(o).astype(jnp.uint32)) + iota_u
                bits = _threefry_bits(flat)
                u = _bits_to_uniform(bits)
                y = _neg_ln(u)
                num = jnp.exp(l)
                upd = num * bd > bn * y
                bn = jnp.where(upd, num, bn)
                bd = jnp.where(upd, y, bd)
                bi = jnp.where(upd, colv, bi)
                blg = jnp.where(upd, l, blg)
            return bn, bd, bi, blg

        init = (jnp.zeros(16, jnp.float32), jnp.ones(16, jnp.float32),
                jnp.zeros(16, jnp.int32), jnp.zeros(16, jnp.float32))
        bn, bd, bi, blg = lax.fori_loop(0, _G // 32, col_body, init)
        bl_vmem[r] = blg
        bi_vmem[r] = bi
        return 0

    lax.fori_loop(0, _B, row_body, 0)

    cpl = pltpu.make_async_copy(bl_vmem, ol_ref.at[t], sem_l)
    cpl.start()
    cpi = pltpu.make_async_copy(bi_vmem, oi_ref.at[t], sem_i)
    cpi.start()
    cpl.wait()
    cpi.wait()


def _sc_candidates(logits):
    fn = pl.kernel(
        _sc_body,
        out_type=(
            jax.ShapeDtypeStruct((_NSUB, _B, 16), jnp.float32),
            jax.ShapeDtypeStruct((_NSUB, _B, 16), jnp.int32),
        ),
        mesh=plsc.VectorSubcoreMesh(core_axis_name="c", subcore_axis_name="s",
                                    num_cores=2, num_subcores=16),
        scratch_types=(
            pltpu.VMEM((_B, _G), jnp.float32),
            pltpu.VMEM((_B, 16), jnp.float32),
            pltpu.VMEM((_B, 16), jnp.int32),
            pltpu.SemaphoreType.DMA,
            pltpu.SemaphoreType.DMA,
            pltpu.SemaphoreType.DMA,
        ),
    )
    return fn(logits)


def _merge_kernel(cl_ref, ci_ref, tm_ref, ti_ref, o_ref):
    ci = ci_ref[...]
    row = lax.broadcasted_iota(jnp.uint32, ci.shape, 1)
    flat = row * jnp.uint32(_V) + ci.astype(jnp.uint32)
    bits = _threefry_bits(flat)
    u = _bits_to_uniform(bits)
    gumbel = -jnp.log(-jnp.log(u))
    sc = cl_ref[...] + gumbel

    run_m = jnp.full((_B, 16), -jnp.inf, jnp.float32)
    run_i = jnp.zeros((_B, 16), jnp.int32)
    for t in range(_NSUB):
        st = sc[t]
        it = ci[t]
        upd = st > run_m
        run_i = jnp.where(upd, it, run_i)
        run_m = jnp.maximum(run_m, st)

    m = jnp.max(run_m, axis=1, keepdims=True)
    cand = jnp.where(run_m == m, run_i, jnp.int32(0x7FFFFFFF))
    ii = jnp.min(cand, axis=1, keepdims=True)
    better = m >= tm_ref[...]
    o_ref[...] = jnp.where(better, ii, ti_ref[...])


def kernel(logits):
    tc_idx, tc_max = pl.pallas_call(
        _tc_kernel,
        grid=(pl.cdiv(_C_TC, _BLK),),
        in_specs=[pl.BlockSpec((_B, _BLK), lambda b: (0, b + _C_SC // _BLK))],
        out_specs=(pl.BlockSpec((_B, 1), lambda b: (0, 0)),
                   pl.BlockSpec((_B, 1), lambda b: (0, 0))),
        out_shape=(jax.ShapeDtypeStruct((_B, 1), jnp.int32),
                   jax.ShapeDtypeStruct((_B, 1), jnp.float32)),
        scratch_shapes=[
            pltpu.VMEM((_B, 128), jnp.float32),
            pltpu.VMEM((_B, 128), jnp.int32),
        ],
    )(logits)

    sc_l, sc_i = _sc_candidates(logits)

    out = pl.pallas_call(
        _merge_kernel,
        out_shape=jax.ShapeDtypeStruct((_B, 1), jnp.int32),
    )(sc_l, sc_i, tc_max, tc_idx)
    return out[:, 0].astype(jnp.int64)


# speedup vs baseline: 1.1133x; 1.0511x over previous
import jax
import jax.numpy as jnp
from jax import lax
from jax.experimental import pallas as pl
from jax.experimental.pallas import tpu as pltpu
from jax.experimental.pallas import tpu_sc as plsc

_B = 64
_V = 100000

_G = 768
_NSUB = 32
_C_SC = _G * _NSUB
_C_TC = _V - _C_SC

_BLK = 8192
_CHUNK = 2048

_K0 = 0
_K1 = 42
_K2 = _K0 ^ _K1 ^ 0x1BD11BDA

_ROTS = ((13, 15, 26, 6), (17, 29, 16, 24))
_SCHED = ((_K1, _K2), (_K2, _K0), (_K0, _K1), (_K1, _K2), (_K2, _K0))

_LN2 = 0.6931471805599453


def _threefry_bits(x1):
    x1 = x1 + jnp.uint32(_K1)
    x0 = x1
    x1 = ((x1 << jnp.uint32(13)) | (x1 >> jnp.uint32(19))) ^ x0
    first = True
    for i in range(5):
        for r in _ROTS[i % 2]:
            if first:
                first = False
                continue
            x0 = x0 + x1
            x1 = (x1 << jnp.uint32(r)) | (x1 >> jnp.uint32(32 - r))
            x1 = x1 ^ x0
        ka, kb = _SCHED[i]
        x0 = x0 + jnp.uint32(ka)
        x1 = x1 + jnp.uint32(kb) + jnp.uint32(i + 1)
    return x0 ^ x1


def _bits_to_uniform(bits):
    fbits = (bits >> jnp.uint32(9)) | jnp.uint32(0x3F800000)
    u = lax.bitcast_convert_type(fbits, jnp.float32) - 1.0
    return jnp.maximum(u, jnp.float32(1e-20))


def _tc_kernel(x_ref, oi_ref, om_ref, max_ref, idx_ref):
    b = pl.program_id(0)
    nb = pl.num_programs(0)

    @pl.when(b == 0)
    def _init():
        max_ref[...] = jnp.full_like(max_ref[...], -jnp.inf)
        idx_ref[...] = jnp.zeros_like(idx_ref[...])

    acc_m = max_ref[...]
    acc_i = idx_ref[...]
    shp = (_B, _CHUNK)
    base2d = (lax.broadcasted_iota(jnp.uint32, shp, 0) * jnp.uint32(_V)
              + lax.broadcasted_iota(jnp.uint32, shp, 1))
    iota2d = lax.broadcasted_iota(jnp.int32, shp, 1)
    for c in range(_BLK // _CHUNK):
        off = b * _BLK + c * _CHUNK + _C_SC
        col = iota2d + off
        flat = base2d + jnp.uint32(off)

        bits = _threefry_bits(flat)
        u = _bits_to_uniform(bits)
        gumbel = -jnp.log(-jnp.log(u))

        pert = x_ref[:, c * _CHUNK:(c + 1) * _CHUNK] + gumbel
        pert = jnp.where(col < _V, pert, -jnp.inf)

        for s in range(_CHUNK // 128):
            tile = pert[:, s * 128:(s + 1) * 128]
            itile = col[:, s * 128:(s + 1) * 128]
            upd = tile > acc_m
            acc_i = jnp.where(upd, itile, acc_i)
            acc_m = jnp.maximum(acc_m, tile)
    max_ref[...] = acc_m
    idx_ref[...] = acc_i

    @pl.when(b == nb - 1)
    def _done():
        acc_mt = acc_m.T
        acc_it = acc_i.T
        m = jnp.max(acc_mt, axis=0, keepdims=True)
        cand = jnp.where(acc_mt == m, acc_it, jnp.int32(0x7FFFFFFF))
        oi_ref[...] = jnp.min(cand, axis=0, keepdims=True)
        om_ref[...] = m


def _neg_ln(u):
    bits = lax.bitcast_convert_type(u, jnp.uint32)
    e = (bits >> jnp.uint32(23)).astype(jnp.int32) - 127
    m = lax.bitcast_convert_type(
        (bits & jnp.uint32(0x7FFFFF)) | jnp.uint32(0x3F800000), jnp.float32)
    big = m > jnp.float32(1.4142135)
    m = jnp.where(big, m * jnp.float32(0.5), m)
    e = e + jnp.where(big, jnp.int32(1), jnp.int32(0))
    ef = e.astype(jnp.float32)
    z = (m - 1.0) / (m + 1.0)
    z2 = z * z
    p = jnp.float32(2 / 9)
    for cc in (2 / 7, 2 / 5, 2 / 3, 2.0):
        p = p * z2 + jnp.float32(cc)
    lnm = z * p
    return -(ef * jnp.float32(_LN2) + lnm)


def _sc_body(x_ref, ol_ref, oi_ref,
             stripe, bl_vmem, bi_vmem, sem_in, sem_l, sem_i):
    c = lax.axis_index("c")
    s = lax.axis_index("s")
    t = c * 16 + s
    col0 = t * _G

    cp = pltpu.make_async_copy(x_ref.at[:, pl.ds(col0, _G)], stripe, sem_in)
    cp.start()
    cp.wait()

    iota_i = lax.iota(jnp.int32, 16)
    iota_u = lax.iota(jnp.uint32, 16)
    _dnums = lax.GatherDimensionNumbers(
        offset_dims=(), collapsed_slice_dims=(0,), start_index_map=(0,))

    def _shuf(x, perm):
        return lax.gather(x, perm[:, None], _dnums, (1,),
                          mode=lax.GatherScatterMode.PROMISE_IN_BOUNDS)

    perms = [iota_i ^ jnp.int32(k) for k in (8, 4, 2, 1)]

    def row_body(r, _):
        base_u = jnp.uint32(r * _V + col0)

        def col_body(v, carry):
            bn, bd, bi, blg = carry
            for k in range(2):
                o = v * 32 + k * 16
                l = stripe[r, pl.ds(o, 16)]
                colv = (col0 + o) + iota_i
                flat = (base_u + jnp.uint32(o).astype(jnp.uint32)) + iota_u
                bits = _threefry_bits(flat)
                u = _bits_to_uniform(bits)
                y = _neg_ln(u)
                num = jnp.exp(l)
                upd = num * bd > bn * y
                bn = jnp.where(upd, num, bn)
                bd = jnp.where(upd, y, bd)
                bi = jnp.where(upd, colv, bi)
                blg = jnp.where(upd, l, blg)
            return bn, bd, bi, blg

        init = (jnp.zeros(16, jnp.float32), jnp.ones(16, jnp.float32),
                jnp.zeros(16, jnp.int32), jnp.zeros(16, jnp.float32))
        bn, bd, bi, blg = lax.fori_loop(0, _G // 32, col_body, init)

        w = bn / bd
        sm = w
        for pm in perms:
            sm = jnp.maximum(sm, _shuf(sm, pm))
        cand = jnp.where(w == sm, bi, jnp.int32(0x7FFFFFFF))
        for pm in perms:
            cand = jnp.minimum(cand, _shuf(cand, pm))
        lgv = jnp.where(bi == cand, blg, jnp.float32(jnp.inf))
        for pm in perms:
            lgv = jnp.minimum(lgv, _shuf(lgv, pm))

        g16 = (r // 16) * 16
        q = r % 16
        pend_l = bl_vmem[0, pl.ds(g16, 16)]
        bl_vmem[0, pl.ds(g16, 16)] = jnp.where(iota_i == q, lgv, pend_l)
        pend_i = bi_vmem[0, pl.ds(g16, 16)]
        bi_vmem[0, pl.ds(g16, 16)] = jnp.where(iota_i == q, cand, pend_i)
        return 0

    lax.fori_loop(0, _B, row_body, 0)

    cpl = pltpu.make_async_copy(bl_vmem, ol_ref.at[t], sem_l)
    cpl.start()
    cpi = pltpu.make_async_copy(bi_vmem, oi_ref.at[t], sem_i)
    cpi.start()
    cpl.wait()
    cpi.wait()


def _sc_candidates(logits):
    fn = pl.kernel(
        _sc_body,
        out_type=(
            jax.ShapeDtypeStruct((_NSUB, 1, _B), jnp.float32),
            jax.ShapeDtypeStruct((_NSUB, 1, _B), jnp.int32),
        ),
        mesh=plsc.VectorSubcoreMesh(core_axis_name="c", subcore_axis_name="s",
                                    num_cores=2, num_subcores=16),
        scratch_types=(
            pltpu.VMEM((_B, _G), jnp.float32),
            pltpu.VMEM((1, _B), jnp.float32),
            pltpu.VMEM((1, _B), jnp.int32),
            pltpu.SemaphoreType.DMA,
            pltpu.SemaphoreType.DMA,
            pltpu.SemaphoreType.DMA,
        ),
    )
    return fn(logits)


def _merge_kernel(cl_ref, ci_ref, tm_ref, ti_ref, o_ref):
    ci = ci_ref[...]
    row = lax.broadcasted_iota(jnp.uint32, ci.shape, 1)
    flat = row * jnp.uint32(_V) + ci.astype(jnp.uint32)
    bits = _threefry_bits(flat)
    u = _bits_to_uniform(bits)
    gumbel = -jnp.log(-jnp.log(u))
    sc = cl_ref[...] + gumbel

    m = jnp.max(sc, axis=0, keepdims=True)
    cand = jnp.where(sc == m, ci, jnp.int32(0x7FFFFFFF))
    ii = jnp.min(cand, axis=0, keepdims=True)
    better = m >= tm_ref[...]
    o_ref[...] = jnp.where(better, ii, ti_ref[...])


def kernel(logits):
    tc_idx, tc_max = pl.pallas_call(
        _tc_kernel,
        grid=(pl.cdiv(_C_TC, _BLK),),
        in_specs=[pl.BlockSpec((_B, _BLK), lambda b: (0, b + _C_SC // _BLK))],
        out_specs=(pl.BlockSpec((1, _B), lambda b: (0, 0)),
                   pl.BlockSpec((1, _B), lambda b: (0, 0))),
        out_shape=(jax.ShapeDtypeStruct((1, _B), jnp.int32),
                   jax.ShapeDtypeStruct((1, _B), jnp.float32)),
        scratch_shapes=[
            pltpu.VMEM((_B, 128), jnp.float32),
            pltpu.VMEM((_B, 128), jnp.int32),
        ],
    )(logits)

    sc_l, sc_i = _sc_candidates(logits)
    sc_l = sc_l.reshape(_NSUB, _B)
    sc_i = sc_i.reshape(_NSUB, _B)

    out = pl.pallas_call(
        _merge_kernel,
        out_shape=jax.ShapeDtypeStruct((1, _B), jnp.int32),
    )(sc_l, sc_i, tc_max, tc_idx)
    return out[0].astype(jnp.int64)
